# Initial kernel scaffold; baseline (speedup 1.0000x reference)
import jax
import jax.numpy as jnp
from jax.experimental import pallas as pl


def kernel(x, edge_index, W1, b1, W2, b2):
    raise NotImplementedError("write your pallas kernel here")


# speedup vs baseline: 123.1433x; 123.1433x over previous
import functools

import jax
import jax.numpy as jnp
from jax import lax
from jax.experimental import pallas as pl
from jax.experimental.pallas import tpu as pltpu
from jax.experimental.pallas import tpu_sc as plsc

N = 100000
E = 6400000
NC = 2
NS = 16
NW = NC * NS
STR = 6272
N_PAD = NS * STR
EPT = E // NW
W = 4000
NWIN = EPT // W


def _mesh():
    return plsc.VectorSubcoreMesh(core_axis_name="c", subcore_axis_name="s")


def _zero_fill(buf, n):
    def body(i, _):
        buf[pl.ds(i * 16, 16)] = jnp.zeros((16,), jnp.float32)
        return 0
    lax.fori_loop(0, n // 16, body, 0)


def _sc_degree(dst):
    @functools.partial(
        pl.kernel,
        out_type=jax.ShapeDtypeStruct((NC, N_PAD), jnp.float32),
        mesh=_mesh(),
        scratch_types=[
            pltpu.VMEM_SHARED((N_PAD,), jnp.float32),
            pltpu.VMEM((W,), jnp.int32),
            pltpu.VMEM((W,), jnp.float32),
            pltpu.VMEM((STR,), jnp.float32),
        ],
    )
    def k(dst_hbm, out_hbm, acc_sh, dbuf, ones_v, zbuf):
        cid = lax.axis_index("c")
        sid = lax.axis_index("s")
        gwid = cid * NS + sid

        _zero_fill(zbuf, STR)

        def init_ones(i, _):
            ones_v[pl.ds(i * 16, 16)] = jnp.ones((16,), jnp.float32)
            return 0
        lax.fori_loop(0, W // 16, init_ones, 0)

        pltpu.sync_copy(zbuf, acc_sh.at[pl.ds(sid * STR, STR)])
        plsc.subcore_barrier()

        def body(w, _):
            base = gwid * EPT + w * W
            pltpu.sync_copy(dst_hbm.at[pl.ds(base, W)], dbuf)
            pltpu.sync_copy(ones_v, acc_sh.at[dbuf], add=True)
            return 0
        lax.fori_loop(0, NWIN, body, 0)

        plsc.subcore_barrier()
        pltpu.sync_copy(acc_sh.at[pl.ds(sid * STR, STR)],
                        out_hbm.at[cid, pl.ds(sid * STR, STR)])

    return k(dst)


def _sc_gather_scatter(g, src, dst, d):
    scratch = (
        [pltpu.VMEM_SHARED((N_PAD,), jnp.float32) for _ in range(2 * d)]
        + [
            pltpu.VMEM((W,), jnp.int32),
            pltpu.VMEM((W,), jnp.int32),
            pltpu.VMEM((W,), jnp.float32),
            pltpu.VMEM((STR,), jnp.float32),
        ]
    )

    @functools.partial(
        pl.kernel,
        out_type=jax.ShapeDtypeStruct((NC, d, N_PAD), jnp.float32),
        mesh=_mesh(),
        scratch_types=scratch,
    )
    def k(g_hbm, src_hbm, dst_hbm, out_hbm, *refs):
        tabs = refs[:d]
        accs = refs[d:2 * d]
        sbuf, dbuf, vbuf, zbuf = refs[2 * d:]
        cid = lax.axis_index("c")
        sid = lax.axis_index("s")
        gwid = cid * NS + sid

        _zero_fill(zbuf, STR)
        for p in range(d):
            pltpu.sync_copy(g_hbm.at[p, pl.ds(sid * STR, STR)],
                            tabs[p].at[pl.ds(sid * STR, STR)])
            pltpu.sync_copy(zbuf, accs[p].at[pl.ds(sid * STR, STR)])
        plsc.subcore_barrier()

        def body(w, _):
            base = gwid * EPT + w * W
            pltpu.sync_copy(src_hbm.at[pl.ds(base, W)], sbuf)
            pltpu.sync_copy(dst_hbm.at[pl.ds(base, W)], dbuf)
            for p in range(d):
                pltpu.sync_copy(tabs[p].at[sbuf], vbuf)
                pltpu.sync_copy(vbuf, accs[p].at[dbuf], add=True)
            return 0
        lax.fori_loop(0, NWIN, body, 0)

        plsc.subcore_barrier()
        for p in range(d):
            pltpu.sync_copy(accs[p].at[pl.ds(sid * STR, STR)],
                            out_hbm.at[cid, p, pl.ds(sid * STR, STR)])

    return k(g, src, dst)


def _tc_prep(degp, xT):
    def body(degp_ref, xT_ref, dinv_ref, g1_ref):
        deg = degp_ref[0:1, :] + degp_ref[1:2, :] + 1.0
        dinv = lax.rsqrt(deg)
        dinv_ref[...] = dinv
        g1_ref[...] = xT_ref[...] * dinv

    return pl.pallas_call(
        body,
        out_shape=[
            jax.ShapeDtypeStruct((1, N_PAD), jnp.float32),
            jax.ShapeDtypeStruct((2, N_PAD), jnp.float32),
        ],
    )(degp, xT)


def _tc_layer(pp, g, dinv, w, b, d_in, d_out, relu):
    def body(pp_ref, g_ref, dinv_ref, w_ref, b_ref, out_ref):
        t = pp_ref[0] + pp_ref[1] + g_ref[...]
        dinv = dinv_ref[...]
        for j in range(d_out):
            acc = t[0:1, :] * w_ref[0, j]
            for i in range(1, d_in):
                acc = acc + t[i:i + 1, :] * w_ref[i, j]
            row = dinv * acc + b_ref[j]
            if relu:
                row = jnp.maximum(row, 0.0) * dinv
            out_ref[pl.ds(j, 1), :] = row

    return pl.pallas_call(
        body,
        in_specs=[
            pl.BlockSpec(),
            pl.BlockSpec(),
            pl.BlockSpec(),
            pl.BlockSpec(memory_space=pltpu.SMEM),
            pl.BlockSpec(memory_space=pltpu.SMEM),
        ],
        out_shape=jax.ShapeDtypeStruct((d_out, N_PAD), jnp.float32),
    )(pp, g, dinv, w, b)


def kernel(x, edge_index, W1, b1, W2, b2):
    src = edge_index[0]
    dst = edge_index[1]
    xT = jnp.zeros((2, N_PAD), jnp.float32).at[:, :N].set(x.T)

    degp = _sc_degree(dst)
    dinv, g1 = _tc_prep(degp, xT)
    sp = _sc_gather_scatter(g1, src, dst, 2)
    g2 = _tc_layer(sp, g1, dinv, W1, b1, 2, 4, relu=True)
    tp = _sc_gather_scatter(g2, src, dst, 4)
    outT = _tc_layer(tp, g2, dinv, W2, b2, 4, 2, relu=False)
    return outT[:, :N].T


# speedup vs baseline: 162.1216x; 1.3165x over previous
import functools

import jax
import jax.numpy as jnp
from jax import lax
from jax.experimental import pallas as pl
from jax.experimental.pallas import tpu as pltpu
from jax.experimental.pallas import tpu_sc as plsc

N = 100000
E = 6400000
NC = 2
NS = 16
NW = NC * NS
STR = 6272
N_PAD = NS * STR
EPT = E // NW
W = 20000
NWIN = EPT // W


def _mesh():
    return plsc.VectorSubcoreMesh(core_axis_name="c", subcore_axis_name="s")


def _zero_fill(buf, n):
    def body(i, _):
        buf[pl.ds(i * 16, 16)] = jnp.zeros((16,), jnp.float32)
        return 0
    lax.fori_loop(0, n // 16, body, 0)


def _sc_degree(dst):
    @functools.partial(
        pl.kernel,
        out_type=jax.ShapeDtypeStruct((NC, N_PAD), jnp.float32),
        mesh=_mesh(),
        scratch_types=[
            pltpu.VMEM_SHARED((N_PAD,), jnp.float32),
            pltpu.VMEM((W,), jnp.int32),
            pltpu.VMEM((W,), jnp.float32),
            pltpu.VMEM((STR,), jnp.float32),
        ],
    )
    def k(dst_hbm, out_hbm, acc_sh, dbuf, ones_v, zbuf):
        cid = lax.axis_index("c")
        sid = lax.axis_index("s")
        gwid = cid * NS + sid

        _zero_fill(zbuf, STR)

        def init_ones(i, _):
            ones_v[pl.ds(i * 16, 16)] = jnp.ones((16,), jnp.float32)
            return 0
        lax.fori_loop(0, W // 16, init_ones, 0)

        pltpu.sync_copy(zbuf, acc_sh.at[pl.ds(sid * STR, STR)])
        plsc.subcore_barrier()

        def body(w, _):
            base = gwid * EPT + w * W
            pltpu.sync_copy(dst_hbm.at[pl.ds(base, W)], dbuf)
            pltpu.sync_copy(ones_v, acc_sh.at[dbuf], add=True)
            return 0
        lax.fori_loop(0, NWIN, body, 0)

        plsc.subcore_barrier()
        pltpu.sync_copy(acc_sh.at[pl.ds(sid * STR, STR)],
                        out_hbm.at[cid, pl.ds(sid * STR, STR)])

    return k(dst)


def _sc_gather_scatter(g, src, dst, d):
    scratch = (
        [pltpu.VMEM_SHARED((N_PAD,), jnp.float32) for _ in range(2 * d)]
        + [
            pltpu.VMEM((W,), jnp.int32),
            pltpu.VMEM((W,), jnp.int32),
            pltpu.VMEM((W,), jnp.float32),
            pltpu.VMEM((STR,), jnp.float32),
        ]
    )

    @functools.partial(
        pl.kernel,
        out_type=jax.ShapeDtypeStruct((NC, d, N_PAD), jnp.float32),
        mesh=_mesh(),
        scratch_types=scratch,
    )
    def k(g_hbm, src_hbm, dst_hbm, out_hbm, *refs):
        tabs = refs[:d]
        accs = refs[d:2 * d]
        sbuf, dbuf, vbuf, zbuf = refs[2 * d:]
        cid = lax.axis_index("c")
        sid = lax.axis_index("s")
        gwid = cid * NS + sid

        _zero_fill(zbuf, STR)
        for p in range(d):
            pltpu.sync_copy(g_hbm.at[p, pl.ds(sid * STR, STR)],
                            tabs[p].at[pl.ds(sid * STR, STR)])
            pltpu.sync_copy(zbuf, accs[p].at[pl.ds(sid * STR, STR)])
        plsc.subcore_barrier()

        def body(w, _):
            base = gwid * EPT + w * W
            pltpu.sync_copy(src_hbm.at[pl.ds(base, W)], sbuf)
            pltpu.sync_copy(dst_hbm.at[pl.ds(base, W)], dbuf)
            for p in range(d):
                pltpu.sync_copy(tabs[p].at[sbuf], vbuf)
                pltpu.sync_copy(vbuf, accs[p].at[dbuf], add=True)
            return 0
        lax.fori_loop(0, NWIN, body, 0)

        plsc.subcore_barrier()
        for p in range(d):
            pltpu.sync_copy(accs[p].at[pl.ds(sid * STR, STR)],
                            out_hbm.at[cid, p, pl.ds(sid * STR, STR)])

    return k(g, src, dst)


def _tc_prep(degp, xT):
    def body(degp_ref, xT_ref, dinv_ref, g1_ref):
        deg = degp_ref[0:1, :] + degp_ref[1:2, :] + 1.0
        dinv = lax.rsqrt(deg)
        dinv_ref[...] = dinv
        g1_ref[...] = xT_ref[...] * dinv

    return pl.pallas_call(
        body,
        out_shape=[
            jax.ShapeDtypeStruct((1, N_PAD), jnp.float32),
            jax.ShapeDtypeStruct((2, N_PAD), jnp.float32),
        ],
    )(degp, xT)


def _tc_layer(pp, g, dinv, w, b, d_in, d_out, relu):
    def body(pp_ref, g_ref, dinv_ref, w_ref, b_ref, out_ref):
        t = pp_ref[0] + pp_ref[1] + g_ref[...]
        dinv = dinv_ref[...]
        for j in range(d_out):
            acc = t[0:1, :] * w_ref[0, j]
            for i in range(1, d_in):
                acc = acc + t[i:i + 1, :] * w_ref[i, j]
            row = dinv * acc + b_ref[j]
            if relu:
                row = jnp.maximum(row, 0.0) * dinv
            out_ref[pl.ds(j, 1), :] = row

    return pl.pallas_call(
        body,
        in_specs=[
            pl.BlockSpec(),
            pl.BlockSpec(),
            pl.BlockSpec(),
            pl.BlockSpec(memory_space=pltpu.SMEM),
            pl.BlockSpec(memory_space=pltpu.SMEM),
        ],
        out_shape=jax.ShapeDtypeStruct((d_out, N_PAD), jnp.float32),
    )(pp, g, dinv, w, b)


def kernel(x, edge_index, W1, b1, W2, b2):
    src = edge_index[0]
    dst = edge_index[1]
    xT = jnp.zeros((2, N_PAD), jnp.float32).at[:, :N].set(x.T)

    degp = _sc_degree(dst)
    dinv, g1 = _tc_prep(degp, xT)
    sp = _sc_gather_scatter(g1, src, dst, 2)
    g2 = _tc_layer(sp, g1, dinv, W1, b1, 2, 4, relu=True)
    tp = _sc_gather_scatter(g2, src, dst, 4)
    outT = _tc_layer(tp, g2, dinv, W2, b2, 4, 2, relu=False)
    return outT[:, :N].T


# speedup vs baseline: 173.7403x; 1.0717x over previous
import functools

import jax
import jax.numpy as jnp
from jax import lax
from jax.experimental import pallas as pl
from jax.experimental.pallas import tpu as pltpu
from jax.experimental.pallas import tpu_sc as plsc

N = 100000
E = 6400000
NC = 2
NS = 16
NW = NC * NS
STR = 6272
N_PAD = NS * STR
EPT = E // NW
WD = 20000
NWIN_D = EPT // WD
WP = 5000
NWIN_P = EPT // WP
NB = 5
NBODY = NWIN_P // NB


def _mesh():
    return plsc.VectorSubcoreMesh(core_axis_name="c", subcore_axis_name="s")


def _zero_fill(buf, n):
    def body(i, _):
        buf[pl.ds(i * 16, 16)] = jnp.zeros((16,), jnp.float32)
        return 0
    lax.fori_loop(0, n // 16, body, 0)


def _sc_degree(dst):
    @functools.partial(
        pl.kernel,
        out_type=jax.ShapeDtypeStruct((NC, N_PAD), jnp.float32),
        mesh=_mesh(),
        scratch_types=[
            pltpu.VMEM_SHARED((N_PAD,), jnp.float32),
            pltpu.VMEM((WD,), jnp.int32),
            pltpu.VMEM((WD,), jnp.int32),
            pltpu.VMEM((WD,), jnp.float32),
            pltpu.SemaphoreType.DMA,
            pltpu.SemaphoreType.DMA,
            pltpu.SemaphoreType.DMA,
        ],
    )
    def k(dst_hbm, out_hbm, acc_sh, dbuf0, dbuf1, ones_v, sem0, sem1, semi):
        cid = lax.axis_index("c")
        sid = lax.axis_index("s")
        gwid = cid * NS + sid
        dbufs = (dbuf0, dbuf1)
        sems = (sem0, sem1)

        _zero_fill(ones_v, STR)
        pltpu.sync_copy(ones_v.at[pl.ds(0, STR)],
                        acc_sh.at[pl.ds(sid * STR, STR)])

        def init_ones(i, _):
            ones_v[pl.ds(i * 16, 16)] = jnp.ones((16,), jnp.float32)
            return 0
        lax.fori_loop(0, WD // 16, init_ones, 0)
        plsc.subcore_barrier()

        pend = [None, None]
        for w in range(NWIN_D):
            par = w % 2
            if pend[par] is not None:
                pend[par].wait()
            base = gwid * EPT + w * WD
            pltpu.async_copy(dst_hbm.at[pl.ds(base, WD)], dbufs[par], semi).wait()
            pend[par] = pltpu.async_copy(ones_v, acc_sh.at[dbufs[par]],
                                         sems[par], add=True)
        for d_ in pend:
            d_.wait()

        plsc.subcore_barrier()
        pltpu.sync_copy(acc_sh.at[pl.ds(sid * STR, STR)],
                        out_hbm.at[cid, pl.ds(sid * STR, STR)])

    return k(dst)


def _sc_gather_scatter(g, src, dst, d):
    scratch = (
        [pltpu.VMEM_SHARED((N_PAD,), jnp.float32) for _ in range(2 * d)]
        + [
            pltpu.VMEM((WP,), jnp.int32),
            pltpu.VMEM((WP,), jnp.int32),
            pltpu.VMEM((WP,), jnp.int32),
        ]
        + [pltpu.VMEM((WP,), jnp.float32) for _ in range(2 * d)]
        + [pltpu.SemaphoreType.DMA] * 4
    )

    @functools.partial(
        pl.kernel,
        out_type=jax.ShapeDtypeStruct((NC, d, N_PAD), jnp.float32),
        mesh=_mesh(),
        scratch_types=scratch,
    )
    def k(g_hbm, src_hbm, dst_hbm, out_hbm, *refs):
        tabs = refs[:d]
        accs = refs[d:2 * d]
        sbuf = refs[2 * d]
        dbufs = (refs[2 * d + 1], refs[2 * d + 2])
        vals = (refs[2 * d + 3:3 * d + 3], refs[3 * d + 3:4 * d + 3])
        sem_g, sem_s0, sem_s1, sem_i = refs[4 * d + 3:]
        sem_s = (sem_s0, sem_s1)
        cid = lax.axis_index("c")
        sid = lax.axis_index("s")
        gwid = cid * NS + sid

        zbuf = vals[0][0]
        _zero_fill(zbuf, STR)
        for p in range(d):
            pltpu.sync_copy(g_hbm.at[p, pl.ds(sid * STR, STR)],
                            tabs[p].at[pl.ds(sid * STR, STR)])
            pltpu.sync_copy(zbuf.at[pl.ds(0, STR)],
                            accs[p].at[pl.ds(sid * STR, STR)])
        plsc.subcore_barrier()

        def body(i, _):
            base0 = gwid * EPT + i * (NB * WP)
            pend = {}
            for kw in range(NB):
                par = kw % 2
                base = base0 + kw * WP
                pltpu.async_copy(src_hbm.at[pl.ds(base, WP)], sbuf,
                                 sem_i).wait()
                if kw - 2 in pend:
                    for d_ in pend.pop(kw - 2):
                        d_.wait()
                pltpu.async_copy(dst_hbm.at[pl.ds(base, WP)], dbufs[par],
                                 sem_i).wait()
                gds = [pltpu.async_copy(tabs[p].at[sbuf], vals[par][p], sem_g)
                       for p in range(d)]
                for d_ in gds:
                    d_.wait()
                pend[kw] = [pltpu.async_copy(vals[par][p],
                                             accs[p].at[dbufs[par]],
                                             sem_s[par], add=True)
                            for p in range(d)]
            for kw in sorted(pend):
                for d_ in pend[kw]:
                    d_.wait()
            return 0
        lax.fori_loop(0, NBODY, body, 0)

        plsc.subcore_barrier()
        for p in range(d):
            pltpu.sync_copy(accs[p].at[pl.ds(sid * STR, STR)],
                            out_hbm.at[cid, p, pl.ds(sid * STR, STR)])

    return k(g, src, dst)


def _tc_prep(degp, xT):
    def body(degp_ref, xT_ref, dinv_ref, g1_ref):
        deg = degp_ref[0:1, :] + degp_ref[1:2, :] + 1.0
        dinv = lax.rsqrt(deg)
        dinv_ref[...] = dinv
        g1_ref[...] = xT_ref[...] * dinv

    return pl.pallas_call(
        body,
        out_shape=[
            jax.ShapeDtypeStruct((1, N_PAD), jnp.float32),
            jax.ShapeDtypeStruct((2, N_PAD), jnp.float32),
        ],
    )(degp, xT)


def _tc_layer(pp, g, dinv, w, b, d_in, d_out, relu):
    def body(pp_ref, g_ref, dinv_ref, w_ref, b_ref, out_ref):
        t = pp_ref[0] + pp_ref[1] + g_ref[...]
        dinv = dinv_ref[...]
        for j in range(d_out):
            acc = t[0:1, :] * w_ref[0, j]
            for i in range(1, d_in):
                acc = acc + t[i:i + 1, :] * w_ref[i, j]
            row = dinv * acc + b_ref[j]
            if relu:
                row = jnp.maximum(row, 0.0) * dinv
            out_ref[pl.ds(j, 1), :] = row

    return pl.pallas_call(
        body,
        in_specs=[
            pl.BlockSpec(),
            pl.BlockSpec(),
            pl.BlockSpec(),
            pl.BlockSpec(memory_space=pltpu.SMEM),
            pl.BlockSpec(memory_space=pltpu.SMEM),
        ],
        out_shape=jax.ShapeDtypeStruct((d_out, N_PAD), jnp.float32),
    )(pp, g, dinv, w, b)


def kernel(x, edge_index, W1, b1, W2, b2):
    src = edge_index[0]
    dst = edge_index[1]
    xT = jnp.zeros((2, N_PAD), jnp.float32).at[:, :N].set(x.T)

    degp = _sc_degree(dst)
    dinv, g1 = _tc_prep(degp, xT)
    sp = _sc_gather_scatter(g1, src, dst, 2)
    g2 = _tc_layer(sp, g1, dinv, W1, b1, 2, 4, relu=True)
    tp = _sc_gather_scatter(g2, src, dst, 4)
    outT = _tc_layer(tp, g2, dinv, W2, b2, 4, 2, relu=False)
    return outT[:, :N].T


# speedup vs baseline: 236.6345x; 1.3620x over previous
import functools

import jax
import jax.numpy as jnp
from jax import lax
from jax.experimental import pallas as pl
from jax.experimental.pallas import tpu as pltpu
from jax.experimental.pallas import tpu_sc as plsc

N = 100000
E = 6400000
NC = 2
NS = 16
NW = NC * NS
STR = 6272
N_PAD = NS * STR
EPT = E // NW
WD = 20000
NWIN_D = EPT // WD
WP = 5000
NWIN_P = EPT // WP
NB = 5
NBODY = NWIN_P // NB


def _mesh():
    return plsc.VectorSubcoreMesh(core_axis_name="c", subcore_axis_name="s")


def _zero_fill(buf, n):
    def body(i, _):
        buf[pl.ds(i * 16, 16)] = jnp.zeros((16,), jnp.float32)
        return 0
    lax.fori_loop(0, n // 16, body, 0)


def _sc_degree(dst):
    @functools.partial(
        pl.kernel,
        out_type=jax.ShapeDtypeStruct((NC, N_PAD), jnp.float32),
        mesh=_mesh(),
        scratch_types=[
            pltpu.VMEM_SHARED((N_PAD,), jnp.float32),
            pltpu.VMEM((WD,), jnp.int32),
            pltpu.VMEM((WD,), jnp.int32),
            pltpu.VMEM((WD,), jnp.float32),
            pltpu.SemaphoreType.DMA,
            pltpu.SemaphoreType.DMA,
            pltpu.SemaphoreType.DMA,
        ],
    )
    def k(dst_hbm, out_hbm, acc_sh, dbuf0, dbuf1, ones_v, sem0, sem1, semi):
        cid = lax.axis_index("c")
        sid = lax.axis_index("s")
        gwid = cid * NS + sid
        dbufs = (dbuf0, dbuf1)
        sems = (sem0, sem1)

        _zero_fill(ones_v, STR)
        pltpu.sync_copy(ones_v.at[pl.ds(0, STR)],
                        acc_sh.at[pl.ds(sid * STR, STR)])

        def init_ones(i, _):
            ones_v[pl.ds(i * 16, 16)] = jnp.ones((16,), jnp.float32)
            return 0
        lax.fori_loop(0, WD // 16, init_ones, 0)
        plsc.subcore_barrier()

        pend = [None, None]
        for w in range(NWIN_D):
            par = w % 2
            if pend[par] is not None:
                pend[par].wait()
            base = gwid * EPT + w * WD
            pltpu.async_copy(dst_hbm.at[pl.ds(base, WD)], dbufs[par], semi).wait()
            pend[par] = pltpu.async_copy(ones_v, acc_sh.at[dbufs[par]],
                                         sems[par], add=True)
        for d_ in pend:
            d_.wait()

        plsc.subcore_barrier()
        pltpu.sync_copy(acc_sh.at[pl.ds(sid * STR, STR)],
                        out_hbm.at[cid, pl.ds(sid * STR, STR)])

    return k(dst)


def _sc_gather_scatter(g, src, dst, d):
    scratch = (
        [pltpu.VMEM_SHARED((N_PAD,), jnp.float32) for _ in range(2 * d)]
        + [
            pltpu.VMEM((WP,), jnp.int32),
            pltpu.VMEM((WP,), jnp.int32),
            pltpu.VMEM((WP,), jnp.int32),
        ]
        + [pltpu.VMEM((WP,), jnp.float32) for _ in range(2 * d)]
        + [pltpu.SemaphoreType.DMA] * 4
    )

    @functools.partial(
        pl.kernel,
        out_type=jax.ShapeDtypeStruct((NC, d, N_PAD), jnp.float32),
        mesh=_mesh(),
        scratch_types=scratch,
    )
    def k(g_hbm, src_hbm, dst_hbm, out_hbm, *refs):
        tabs = refs[:d]
        accs = refs[d:2 * d]
        sbuf = refs[2 * d]
        dbufs = (refs[2 * d + 1], refs[2 * d + 2])
        vals = (refs[2 * d + 3:3 * d + 3], refs[3 * d + 3:4 * d + 3])
        sem_g, sem_s0, sem_s1, sem_i = refs[4 * d + 3:]
        sem_s = (sem_s0, sem_s1)
        cid = lax.axis_index("c")
        sid = lax.axis_index("s")
        gwid = cid * NS + sid

        zbuf = vals[0][0]
        _zero_fill(zbuf, STR)
        for p in range(d):
            pltpu.sync_copy(g_hbm.at[p, pl.ds(sid * STR, STR)],
                            tabs[p].at[pl.ds(sid * STR, STR)])
            pltpu.sync_copy(zbuf.at[pl.ds(0, STR)],
                            accs[p].at[pl.ds(sid * STR, STR)])
        plsc.subcore_barrier()

        def body(i, _):
            base0 = gwid * EPT + i * (NB * WP)
            pend = {}
            for kw in range(NB):
                par = kw % 2
                base = base0 + kw * WP
                pltpu.async_copy(src_hbm.at[pl.ds(base, WP)], sbuf,
                                 sem_i).wait()
                if kw - 2 in pend:
                    for d_ in pend.pop(kw - 2):
                        d_.wait()
                pltpu.async_copy(dst_hbm.at[pl.ds(base, WP)], dbufs[par],
                                 sem_i).wait()
                gds = [pltpu.async_copy(tabs[p].at[sbuf], vals[par][p], sem_g)
                       for p in range(d)]
                for d_ in gds:
                    d_.wait()
                pend[kw] = [pltpu.async_copy(vals[par][p],
                                             accs[p].at[dbufs[par]],
                                             sem_s[par], add=True)
                            for p in range(d)]
            for kw in sorted(pend):
                for d_ in pend[kw]:
                    d_.wait()
            return 0
        lax.fori_loop(0, NBODY, body, 0)

        plsc.subcore_barrier()
        for p in range(d):
            pltpu.sync_copy(accs[p].at[pl.ds(sid * STR, STR)],
                            out_hbm.at[cid, p, pl.ds(sid * STR, STR)])

    return k(g, src, dst)


def _tc_prep(degp, xT):
    def body(degp_ref, xT_ref, dinv_ref, g1_ref):
        deg = degp_ref[0:1, :] + degp_ref[1:2, :] + 1.0
        dinv = lax.rsqrt(deg)
        dinv_ref[...] = dinv
        g1_ref[...] = xT_ref[...] * dinv

    return pl.pallas_call(
        body,
        out_shape=[
            jax.ShapeDtypeStruct((1, N_PAD), jnp.float32),
            jax.ShapeDtypeStruct((2, N_PAD), jnp.float32),
        ],
    )(degp, xT)


def _tc_mid(sp, g1, dinv, w1, b1, w2):
    def body(sp_ref, g_ref, dinv_ref, w1_ref, b1_ref, w2_ref, out_ref):
        t = sp_ref[0] + sp_ref[1] + g_ref[...]
        dinv = dinv_ref[...]
        h = []
        for i in range(4):
            acc = t[0:1, :] * w1_ref[0, i] + t[1:2, :] * w1_ref[1, i]
            h.append(jnp.maximum(dinv * acc + b1_ref[i], 0.0))
        for j in range(2):
            acc = h[0] * w2_ref[0, j]
            for i in range(1, 4):
                acc = acc + h[i] * w2_ref[i, j]
            out_ref[pl.ds(j, 1), :] = dinv * acc

    return pl.pallas_call(
        body,
        in_specs=[
            pl.BlockSpec(),
            pl.BlockSpec(),
            pl.BlockSpec(),
            pl.BlockSpec(memory_space=pltpu.SMEM),
            pl.BlockSpec(memory_space=pltpu.SMEM),
            pl.BlockSpec(memory_space=pltpu.SMEM),
        ],
        out_shape=jax.ShapeDtypeStruct((2, N_PAD), jnp.float32),
    )(sp, g1, dinv, w1, b1, w2)


def _tc_final(tp, g2p, dinv, b2):
    def body(tp_ref, g_ref, dinv_ref, b2_ref, out_ref):
        t = tp_ref[0] + tp_ref[1] + g_ref[...]
        dinv = dinv_ref[...]
        for j in range(2):
            out_ref[pl.ds(j, 1), :] = dinv * t[j:j + 1, :] + b2_ref[j]

    return pl.pallas_call(
        body,
        in_specs=[
            pl.BlockSpec(),
            pl.BlockSpec(),
            pl.BlockSpec(),
            pl.BlockSpec(memory_space=pltpu.SMEM),
        ],
        out_shape=jax.ShapeDtypeStruct((2, N_PAD), jnp.float32),
    )(tp, g2p, dinv, b2)


def kernel(x, edge_index, W1, b1, W2, b2):
    src = edge_index[0]
    dst = edge_index[1]
    xT = jnp.zeros((2, N_PAD), jnp.float32).at[:, :N].set(x.T)

    degp = _sc_degree(dst)
    dinv, g1 = _tc_prep(degp, xT)
    sp = _sc_gather_scatter(g1, src, dst, 2)
    g2p = _tc_mid(sp, g1, dinv, W1, b1, W2)
    tp = _sc_gather_scatter(g2p, src, dst, 2)
    outT = _tc_final(tp, g2p, dinv, b2)
    return outT[:, :N].T


# speedup vs baseline: 246.1686x; 1.0403x over previous
import functools

import jax
import jax.numpy as jnp
from jax import lax
from jax.experimental import pallas as pl
from jax.experimental.pallas import tpu as pltpu
from jax.experimental.pallas import tpu_sc as plsc

N = 100000
E = 6400000
NC = 2
NS = 16
NW = NC * NS
STR = 6272
N_PAD = NS * STR
EPT = E // NW
WD = 20000
NWIN_D = EPT // WD
WP = 10000
NWIN_P = EPT // WP
NB = 5
NBODY = NWIN_P // NB


def _mesh():
    return plsc.VectorSubcoreMesh(core_axis_name="c", subcore_axis_name="s")


def _zero_fill(buf, n):
    def body(i, _):
        buf[pl.ds(i * 16, 16)] = jnp.zeros((16,), jnp.float32)
        return 0
    lax.fori_loop(0, n // 16, body, 0)


def _sc_degree(dst):
    @functools.partial(
        pl.kernel,
        out_type=jax.ShapeDtypeStruct((NC, N_PAD), jnp.float32),
        mesh=_mesh(),
        scratch_types=[
            pltpu.VMEM_SHARED((N_PAD,), jnp.float32),
            pltpu.VMEM((WD,), jnp.int32),
            pltpu.VMEM((WD,), jnp.int32),
            pltpu.VMEM((WD,), jnp.float32),
            pltpu.SemaphoreType.DMA,
            pltpu.SemaphoreType.DMA,
            pltpu.SemaphoreType.DMA,
        ],
    )
    def k(dst_hbm, out_hbm, acc_sh, dbuf0, dbuf1, ones_v, sem0, sem1, semi):
        cid = lax.axis_index("c")
        sid = lax.axis_index("s")
        gwid = cid * NS + sid
        dbufs = (dbuf0, dbuf1)
        sems = (sem0, sem1)

        _zero_fill(ones_v, STR)
        pltpu.sync_copy(ones_v.at[pl.ds(0, STR)],
                        acc_sh.at[pl.ds(sid * STR, STR)])

        def init_ones(i, _):
            ones_v[pl.ds(i * 16, 16)] = jnp.ones((16,), jnp.float32)
            return 0
        lax.fori_loop(0, WD // 16, init_ones, 0)
        plsc.subcore_barrier()

        pend = [None, None]
        for w in range(NWIN_D):
            par = w % 2
            if pend[par] is not None:
                pend[par].wait()
            base = gwid * EPT + w * WD
            pltpu.async_copy(dst_hbm.at[pl.ds(base, WD)], dbufs[par], semi).wait()
            pend[par] = pltpu.async_copy(ones_v, acc_sh.at[dbufs[par]],
                                         sems[par], add=True)
        for d_ in pend:
            d_.wait()

        plsc.subcore_barrier()
        pltpu.sync_copy(acc_sh.at[pl.ds(sid * STR, STR)],
                        out_hbm.at[cid, pl.ds(sid * STR, STR)])

    return k(dst)


def _sc_gather_scatter(g, src, dst, d):
    scratch = (
        [pltpu.VMEM_SHARED((N_PAD,), jnp.float32) for _ in range(2 * d)]
        + [
            pltpu.VMEM((WP,), jnp.int32),
            pltpu.VMEM((WP,), jnp.int32),
            pltpu.VMEM((WP,), jnp.int32),
        ]
        + [pltpu.VMEM((WP,), jnp.float32) for _ in range(2 * d)]
        + [pltpu.SemaphoreType.DMA] * 4
    )

    @functools.partial(
        pl.kernel,
        out_type=jax.ShapeDtypeStruct((NC, d, N_PAD), jnp.float32),
        mesh=_mesh(),
        scratch_types=scratch,
    )
    def k(g_hbm, src_hbm, dst_hbm, out_hbm, *refs):
        tabs = refs[:d]
        accs = refs[d:2 * d]
        sbuf = refs[2 * d]
        dbufs = (refs[2 * d + 1], refs[2 * d + 2])
        vals = (refs[2 * d + 3:3 * d + 3], refs[3 * d + 3:4 * d + 3])
        sem_g, sem_s0, sem_s1, sem_i = refs[4 * d + 3:]
        sem_s = (sem_s0, sem_s1)
        cid = lax.axis_index("c")
        sid = lax.axis_index("s")
        gwid = cid * NS + sid

        zbuf = vals[0][0]
        _zero_fill(zbuf, STR)
        for p in range(d):
            pltpu.sync_copy(g_hbm.at[p, pl.ds(sid * STR, STR)],
                            tabs[p].at[pl.ds(sid * STR, STR)])
            pltpu.sync_copy(zbuf.at[pl.ds(0, STR)],
                            accs[p].at[pl.ds(sid * STR, STR)])
        plsc.subcore_barrier()

        def body(i, _):
            base0 = gwid * EPT + i * (NB * WP)
            pend = {}
            for kw in range(NB):
                par = kw % 2
                base = base0 + kw * WP
                pltpu.async_copy(src_hbm.at[pl.ds(base, WP)], sbuf,
                                 sem_i).wait()
                if kw - 2 in pend:
                    for d_ in pend.pop(kw - 2):
                        d_.wait()
                pltpu.async_copy(dst_hbm.at[pl.ds(base, WP)], dbufs[par],
                                 sem_i).wait()
                gds = [pltpu.async_copy(tabs[p].at[sbuf], vals[par][p], sem_g)
                       for p in range(d)]
                for d_ in gds:
                    d_.wait()
                pend[kw] = [pltpu.async_copy(vals[par][p],
                                             accs[p].at[dbufs[par]],
                                             sem_s[par], add=True)
                            for p in range(d)]
            for kw in sorted(pend):
                for d_ in pend[kw]:
                    d_.wait()
            return 0
        lax.fori_loop(0, NBODY, body, 0)

        plsc.subcore_barrier()
        for p in range(d):
            pltpu.sync_copy(accs[p].at[pl.ds(sid * STR, STR)],
                            out_hbm.at[cid, p, pl.ds(sid * STR, STR)])

    return k(g, src, dst)


def _tc_prep(degp, xT):
    def body(degp_ref, xT_ref, dinv_ref, g1_ref):
        deg = degp_ref[0:1, :] + degp_ref[1:2, :] + 1.0
        dinv = lax.rsqrt(deg)
        dinv_ref[...] = dinv
        g1_ref[...] = xT_ref[...] * dinv

    return pl.pallas_call(
        body,
        out_shape=[
            jax.ShapeDtypeStruct((1, N_PAD), jnp.float32),
            jax.ShapeDtypeStruct((2, N_PAD), jnp.float32),
        ],
    )(degp, xT)


def _tc_mid(sp, g1, dinv, w1, b1, w2):
    def body(sp_ref, g_ref, dinv_ref, w1_ref, b1_ref, w2_ref, out_ref):
        t = sp_ref[0] + sp_ref[1] + g_ref[...]
        dinv = dinv_ref[...]
        h = []
        for i in range(4):
            acc = t[0:1, :] * w1_ref[0, i] + t[1:2, :] * w1_ref[1, i]
            h.append(jnp.maximum(dinv * acc + b1_ref[i], 0.0))
        for j in range(2):
            acc = h[0] * w2_ref[0, j]
            for i in range(1, 4):
                acc = acc + h[i] * w2_ref[i, j]
            out_ref[pl.ds(j, 1), :] = dinv * acc

    return pl.pallas_call(
        body,
        in_specs=[
            pl.BlockSpec(),
            pl.BlockSpec(),
            pl.BlockSpec(),
            pl.BlockSpec(memory_space=pltpu.SMEM),
            pl.BlockSpec(memory_space=pltpu.SMEM),
            pl.BlockSpec(memory_space=pltpu.SMEM),
        ],
        out_shape=jax.ShapeDtypeStruct((2, N_PAD), jnp.float32),
    )(sp, g1, dinv, w1, b1, w2)


def _tc_final(tp, g2p, dinv, b2):
    def body(tp_ref, g_ref, dinv_ref, b2_ref, out_ref):
        t = tp_ref[0] + tp_ref[1] + g_ref[...]
        dinv = dinv_ref[...]
        for j in range(2):
            out_ref[pl.ds(j, 1), :] = dinv * t[j:j + 1, :] + b2_ref[j]

    return pl.pallas_call(
        body,
        in_specs=[
            pl.BlockSpec(),
            pl.BlockSpec(),
            pl.BlockSpec(),
            pl.BlockSpec(memory_space=pltpu.SMEM),
        ],
        out_shape=jax.ShapeDtypeStruct((2, N_PAD), jnp.float32),
    )(tp, g2p, dinv, b2)


def kernel(x, edge_index, W1, b1, W2, b2):
    src = edge_index[0]
    dst = edge_index[1]
    xT = jnp.zeros((2, N_PAD), jnp.float32).at[:, :N].set(x.T)

    degp = _sc_degree(dst)
    dinv, g1 = _tc_prep(degp, xT)
    sp = _sc_gather_scatter(g1, src, dst, 2)
    g2p = _tc_mid(sp, g1, dinv, W1, b1, W2)
    tp = _sc_gather_scatter(g2p, src, dst, 2)
    outT = _tc_final(tp, g2p, dinv, b2)
    return outT[:, :N].T


# speedup vs baseline: 275.1227x; 1.1176x over previous
import functools

import jax
import jax.numpy as jnp
from jax import lax
from jax.experimental import pallas as pl
from jax.experimental.pallas import tpu as pltpu
from jax.experimental.pallas import tpu_sc as plsc

N = 100000
E = 6400000
NC = 2
NS = 16
NW = NC * NS
STR = 6272
N_PAD = NS * STR
EPT = E // NW
WD = 20000
NWIN_D = EPT // WD
WP = 10000
NWIN_P = EPT // WP
NB = 5
NBODY = NWIN_P // NB


def _mesh():
    return plsc.VectorSubcoreMesh(core_axis_name="c", subcore_axis_name="s")


def _zero_fill(buf, n):
    def body(i, _):
        buf[pl.ds(i * 16, 16)] = jnp.zeros((16,), jnp.float32)
        return 0
    lax.fori_loop(0, n // 16, body, 0)


def _sc_degree(dst):
    @functools.partial(
        pl.kernel,
        out_type=jax.ShapeDtypeStruct((NC, N_PAD), jnp.float32),
        mesh=_mesh(),
        scratch_types=[
            pltpu.VMEM_SHARED((N_PAD,), jnp.float32),
            pltpu.VMEM((WD,), jnp.int32),
            pltpu.VMEM((WD,), jnp.int32),
            pltpu.VMEM((WD,), jnp.float32),
            pltpu.SemaphoreType.DMA,
            pltpu.SemaphoreType.DMA,
            pltpu.SemaphoreType.DMA,
        ],
    )
    def k(dst_hbm, out_hbm, acc_sh, dbuf0, dbuf1, ones_v, sem0, sem1, semi):
        cid = lax.axis_index("c")
        sid = lax.axis_index("s")
        gwid = cid * NS + sid
        dbufs = (dbuf0, dbuf1)
        sems = (sem0, sem1)

        _zero_fill(ones_v, STR)
        pltpu.sync_copy(ones_v.at[pl.ds(0, STR)],
                        acc_sh.at[pl.ds(sid * STR, STR)])

        def init_ones(i, _):
            ones_v[pl.ds(i * 16, 16)] = jnp.ones((16,), jnp.float32)
            return 0
        lax.fori_loop(0, WD // 16, init_ones, 0)
        plsc.subcore_barrier()

        pend = [None, None]
        for w in range(NWIN_D):
            par = w % 2
            if pend[par] is not None:
                pend[par].wait()
            base = gwid * EPT + w * WD
            pltpu.async_copy(dst_hbm.at[pl.ds(base, WD)], dbufs[par], semi).wait()
            pend[par] = pltpu.async_copy(ones_v, acc_sh.at[dbufs[par]],
                                         sems[par], add=True)
        for d_ in pend:
            d_.wait()

        plsc.subcore_barrier()
        pltpu.sync_copy(acc_sh.at[pl.ds(sid * STR, STR)],
                        out_hbm.at[cid, pl.ds(sid * STR, STR)])

    return k(dst)


def _sc_gather_scatter(g, src, dst, d):
    scratch = (
        [pltpu.VMEM_SHARED((N_PAD,), jnp.float32) for _ in range(2 * d)]
        + [
            pltpu.VMEM((WP,), jnp.int32),
            pltpu.VMEM((WP,), jnp.int32),
            pltpu.VMEM((WP,), jnp.int32),
        ]
        + [pltpu.VMEM((WP,), jnp.float32) for _ in range(2 * d)]
        + [pltpu.SemaphoreType.DMA] * 4
    )

    @functools.partial(
        pl.kernel,
        out_type=jax.ShapeDtypeStruct((NC, d, N_PAD), jnp.float32),
        mesh=_mesh(),
        scratch_types=scratch,
    )
    def k(g_hbm, src_hbm, dst_hbm, out_hbm, *refs):
        tabs = refs[:d]
        accs = refs[d:2 * d]
        sbuf = refs[2 * d]
        dbufs = (refs[2 * d + 1], refs[2 * d + 2])
        vals = (refs[2 * d + 3:3 * d + 3], refs[3 * d + 3:4 * d + 3])
        sem_g, sem_s0, sem_s1, sem_i = refs[4 * d + 3:]
        sem_s = (sem_s0, sem_s1)
        cid = lax.axis_index("c")
        sid = lax.axis_index("s")
        gwid = cid * NS + sid

        zbuf = vals[0][0]
        _zero_fill(zbuf, STR)
        for p in range(d):
            pltpu.sync_copy(g_hbm.at[p, pl.ds(sid * STR, STR)],
                            tabs[p].at[pl.ds(sid * STR, STR)])
            pltpu.sync_copy(zbuf.at[pl.ds(0, STR)],
                            accs[p].at[pl.ds(sid * STR, STR)])
        plsc.subcore_barrier()

        def body(i, _):
            base0 = gwid * EPT + i * (NB * WP)
            pend = {}
            for kw in range(NB):
                par = kw % 2
                base = base0 + kw * WP
                pltpu.async_copy(src_hbm.at[pl.ds(base, WP)], sbuf,
                                 sem_i).wait()
                if kw - 2 in pend:
                    for d_ in pend.pop(kw - 2):
                        d_.wait()
                pltpu.async_copy(dst_hbm.at[pl.ds(base, WP)], dbufs[par],
                                 sem_i).wait()
                gds = [pltpu.async_copy(tabs[p].at[sbuf], vals[par][p], sem_g)
                       for p in range(d)]
                for d_ in gds:
                    d_.wait()
                pend[kw] = [pltpu.async_copy(vals[par][p],
                                             accs[p].at[dbufs[par]],
                                             sem_s[par], add=True)
                            for p in range(d)]
            for kw in sorted(pend):
                for d_ in pend[kw]:
                    d_.wait()
            return 0
        lax.fori_loop(0, NBODY, body, 0)

        plsc.subcore_barrier()
        for p in range(d):
            pltpu.sync_copy(accs[p].at[pl.ds(sid * STR, STR)],
                            out_hbm.at[cid, p, pl.ds(sid * STR, STR)])

    return k(g, src, dst)


def _sc_gather_scatter_packed(gpk, src, dst):
    scratch = [
        pltpu.VMEM_SHARED((N_PAD,), jnp.int32),
        pltpu.VMEM_SHARED((N_PAD,), jnp.float32),
        pltpu.VMEM_SHARED((N_PAD,), jnp.float32),
        pltpu.VMEM((WP,), jnp.int32),
        pltpu.VMEM((WP,), jnp.int32),
        pltpu.VMEM((WP,), jnp.int32),
        pltpu.VMEM((WP,), jnp.int32),
        pltpu.VMEM((WP,), jnp.int32),
        pltpu.VMEM((WP,), jnp.float32),
        pltpu.VMEM((WP,), jnp.float32),
        pltpu.VMEM((WP,), jnp.float32),
        pltpu.VMEM((WP,), jnp.float32),
        pltpu.SemaphoreType.DMA,
        pltpu.SemaphoreType.DMA,
        pltpu.SemaphoreType.DMA,
        pltpu.SemaphoreType.DMA,
    ]

    @functools.partial(
        pl.kernel,
        out_type=jax.ShapeDtypeStruct((NC, 2, N_PAD), jnp.float32),
        mesh=_mesh(),
        scratch_types=scratch,
    )
    def k(gpk_hbm, src_hbm, dst_hbm, out_hbm, tabu, acc0, acc1, sbuf,
          dbuf0, dbuf1, vu0, vu1, vf00, vf01, vf10, vf11,
          sem_g, sem_s0, sem_s1, sem_i):
        accs = (acc0, acc1)
        dbufs = (dbuf0, dbuf1)
        vus = (vu0, vu1)
        vfs = ((vf00, vf01), (vf10, vf11))
        sem_s = (sem_s0, sem_s1)
        cid = lax.axis_index("c")
        sid = lax.axis_index("s")
        gwid = cid * NS + sid
        stripe = pl.ds(sid * STR, STR)

        pltpu.sync_copy(gpk_hbm.at[stripe], tabu.at[stripe])
        _zero_fill(vf00, STR)
        pltpu.sync_copy(vf00.at[pl.ds(0, STR)], acc0.at[stripe])
        pltpu.sync_copy(vf00.at[pl.ds(0, STR)], acc1.at[stripe])
        plsc.subcore_barrier()

        def body(i, _):
            base0 = gwid * EPT + i * (NB * WP)
            pend = {}
            for kw in range(NB):
                par = kw % 2
                base = base0 + kw * WP
                pltpu.async_copy(src_hbm.at[pl.ds(base, WP)], sbuf,
                                 sem_i).wait()
                if kw - 2 in pend:
                    for d_ in pend.pop(kw - 2):
                        d_.wait()
                pltpu.async_copy(dst_hbm.at[pl.ds(base, WP)], dbufs[par],
                                 sem_i).wait()
                pltpu.async_copy(tabu.at[sbuf], vus[par], sem_g).wait()

                def unpack(j, _, par=par):
                    himask = jnp.full((16,), -65536, jnp.int32)
                    u = vus[par][pl.ds(j * 16, 16)]
                    vfs[par][0][pl.ds(j * 16, 16)] = lax.bitcast_convert_type(
                        u & himask, jnp.float32)
                    vfs[par][1][pl.ds(j * 16, 16)] = lax.bitcast_convert_type(
                        u << 16, jnp.float32)
                    return 0
                lax.fori_loop(0, WP // 16, unpack, 0)

                pend[kw] = [pltpu.async_copy(vfs[par][p],
                                             accs[p].at[dbufs[par]],
                                             sem_s[par], add=True)
                            for p in range(2)]
            for kw in sorted(pend):
                for d_ in pend[kw]:
                    d_.wait()
            return 0
        lax.fori_loop(0, NBODY, body, 0)

        plsc.subcore_barrier()
        for p in range(2):
            pltpu.sync_copy(accs[p].at[stripe],
                            out_hbm.at[cid, p, stripe])

    return k(gpk, src, dst)


def _pack_bf16_pair(a, b):
    ia = jax.lax.bitcast_convert_type(a, jnp.int32) + 0x8000
    ib = jax.lax.bitcast_convert_type(b, jnp.int32) + 0x8000
    return (ia & (-65536)) | ((ib >> 16) & 0xFFFF)


def _tc_prep(degp, xT):
    def body(degp_ref, xT_ref, dinv_ref, g1_ref, g1pk_ref):
        deg = degp_ref[0:1, :] + degp_ref[1:2, :] + 1.0
        dinv = lax.rsqrt(deg)
        dinv_ref[...] = dinv
        g1 = xT_ref[...] * dinv
        g1_ref[...] = g1
        g1pk_ref[...] = _pack_bf16_pair(g1[0:1, :], g1[1:2, :])

    return pl.pallas_call(
        body,
        out_shape=[
            jax.ShapeDtypeStruct((1, N_PAD), jnp.float32),
            jax.ShapeDtypeStruct((2, N_PAD), jnp.float32),
            jax.ShapeDtypeStruct((1, N_PAD), jnp.int32),
        ],
    )(degp, xT)


def _tc_mid(sp, g1, dinv, w1, b1, w2):
    def body(sp_ref, g_ref, dinv_ref, w1_ref, b1_ref, w2_ref, out_ref, pk_ref):
        t = sp_ref[0] + sp_ref[1] + g_ref[...]
        dinv = dinv_ref[...]
        h = []
        for i in range(4):
            acc = t[0:1, :] * w1_ref[0, i] + t[1:2, :] * w1_ref[1, i]
            h.append(jnp.maximum(dinv * acc + b1_ref[i], 0.0))
        rows = []
        for j in range(2):
            acc = h[0] * w2_ref[0, j]
            for i in range(1, 4):
                acc = acc + h[i] * w2_ref[i, j]
            rows.append(dinv * acc)
            out_ref[pl.ds(j, 1), :] = rows[j]
        pk_ref[...] = _pack_bf16_pair(rows[0], rows[1])

    return pl.pallas_call(
        body,
        in_specs=[
            pl.BlockSpec(),
            pl.BlockSpec(),
            pl.BlockSpec(),
            pl.BlockSpec(memory_space=pltpu.SMEM),
            pl.BlockSpec(memory_space=pltpu.SMEM),
            pl.BlockSpec(memory_space=pltpu.SMEM),
        ],
        out_shape=[
            jax.ShapeDtypeStruct((2, N_PAD), jnp.float32),
            jax.ShapeDtypeStruct((1, N_PAD), jnp.int32),
        ],
    )(sp, g1, dinv, w1, b1, w2)


def _tc_final(tp, g2p, dinv, b2):
    def body(tp_ref, g_ref, dinv_ref, b2_ref, out_ref):
        t = tp_ref[0] + tp_ref[1] + g_ref[...]
        dinv = dinv_ref[...]
        for j in range(2):
            out_ref[pl.ds(j, 1), :] = dinv * t[j:j + 1, :] + b2_ref[j]

    return pl.pallas_call(
        body,
        in_specs=[
            pl.BlockSpec(),
            pl.BlockSpec(),
            pl.BlockSpec(),
            pl.BlockSpec(memory_space=pltpu.SMEM),
        ],
        out_shape=jax.ShapeDtypeStruct((2, N_PAD), jnp.float32),
    )(tp, g2p, dinv, b2)


def kernel(x, edge_index, W1, b1, W2, b2):
    src = edge_index[0]
    dst = edge_index[1]
    xT = jnp.zeros((2, N_PAD), jnp.float32).at[:, :N].set(x.T)

    degp = _sc_degree(dst)
    dinv, g1, g1pk = _tc_prep(degp, xT)
    sp = _sc_gather_scatter_packed(g1pk.reshape(N_PAD), src, dst)
    g2p, g2pk = _tc_mid(sp, g1, dinv, W1, b1, W2)
    tp = _sc_gather_scatter_packed(g2pk.reshape(N_PAD), src, dst)
    outT = _tc_final(tp, g2p, dinv, b2)
    return outT[:, :N].T
